# Initial kernel scaffold; baseline (speedup 1.0000x reference)
import jax
import jax.numpy as jnp
from jax.experimental import pallas as pl


def kernel(L, mask, inputs, W1, b1, rn, gamma2, beta2, W2, b2):
    raise NotImplementedError("write your pallas kernel here")


# speedup vs baseline: 2.0280x; 2.0280x over previous
import jax
import jax.numpy as jnp
from jax.experimental import pallas as pl

_N = 100000
_C = 128
_BLK = 5000
_EPS = 1e-5


def _elu(x):
    return jnp.where(x > 0, x, jnp.exp(jnp.minimum(x, 0.0)) - 1.0)


def _make_pass(pre_elu: bool, add_res: bool):
    def kfn(*refs):
        if add_res:
            x_ref, w_ref, b_ref, res_ref, y_ref, st_ref = refs
        else:
            x_ref, w_ref, b_ref, y_ref, st_ref = refs
        i = pl.program_id(0)
        x = x_ref[...]
        e = _elu(x) if pre_elu else x
        y = jnp.dot(e, w_ref[...], preferred_element_type=jnp.float32)
        y = y + b_ref[0:1, :]
        if add_res:
            y = y + res_ref[...]
        y_ref[...] = y
        ey = _elu(y)
        s = jnp.sum(ey, axis=0, keepdims=True)
        s2 = jnp.sum(ey * ey, axis=0, keepdims=True)
        st = jnp.concatenate([s, s2, jnp.zeros((6, _C), jnp.float32)], axis=0)

        @pl.when(i == 0)
        def _init():
            st_ref[...] = jnp.zeros_like(st_ref)

        st_ref[...] += st

    return kfn


def _run_pass(x, w, b, res=None, *, pre_elu):
    add_res = res is not None
    in_specs = [
        pl.BlockSpec((_BLK, _C), lambda i: (i, 0)),
        pl.BlockSpec((_C, _C), lambda i: (0, 0)),
        pl.BlockSpec((8, _C), lambda i: (0, 0)),
    ]
    ops = [x, w, jnp.broadcast_to(b[None, :], (8, _C))]
    if add_res:
        in_specs.append(pl.BlockSpec((_BLK, _C), lambda i: (i, 0)))
        ops.append(res)
    y, st = pl.pallas_call(
        _make_pass(pre_elu, add_res),
        grid=(_N // _BLK,),
        in_specs=in_specs,
        out_specs=[
            pl.BlockSpec((_BLK, _C), lambda i: (i, 0)),
            pl.BlockSpec((8, _C), lambda i: (0, 0)),
        ],
        out_shape=[
            jax.ShapeDtypeStruct((_N, _C), jnp.float32),
            jax.ShapeDtypeStruct((8, _C), jnp.float32),
        ],
    )(*ops)
    return y, st


def _fold(st, gamma_lo, beta_lo, wa, extra_b):
    s, s2 = st[0], st[1]
    m = s / _N
    v = s2 / _N - m * m
    scale = gamma_lo * jax.lax.rsqrt(v + _EPS)
    shift = beta_lo - m * scale
    return scale[:, None] * wa, shift @ wa + extra_b


def kernel(L, mask, inputs, W1, b1, rn, gamma2, beta2, W2, b2):
    del L, mask
    x0 = inputs.reshape(_N, _C)
    x, st = _run_pass(x0, W1, b1, pre_elu=False)
    res = x
    for i in range(4):
        for h in range(2):
            g = rn['gamma%d' % h][i]
            bt = rn['beta%d' % h][i]
            W = rn['W%d' % h][i]
            bb = rn['b%d' % h][i]
            w_eff, b_eff = _fold(st, g[:_C], bt[:_C], W[:_C],
                                 bt[_C:] @ W[_C:] + bb)
            if h == 1:
                x, st = _run_pass(x, w_eff, b_eff, res, pre_elu=True)
                res = x
            else:
                x, st = _run_pass(x, w_eff, b_eff, pre_elu=True)
    w_eff, b_eff = _fold(st, gamma2, beta2, W2, b2)
    y, _ = _run_pass(x, w_eff, b_eff, x0, pre_elu=True)
    return y.reshape(1, _N, _C)


# speedup vs baseline: 2.0406x; 1.0062x over previous
import jax
import jax.numpy as jnp
from jax.experimental import pallas as pl

_N = 100000
_C = 128
_BLK = 5000
_EPS = 1e-5


def _elu(x):
    return jnp.where(x > 0, x, jnp.exp(jnp.minimum(x, 0.0)) - 1.0)


def _make_pass(pre_norm: bool, add_res: bool):
    def kfn(*refs):
        if pre_norm:
            x_ref, w_ref, b_ref, st_in_ref, gb_ref, *rest = refs
        else:
            x_ref, w_ref, b_ref, *rest = refs
        if add_res:
            res_ref, y_ref, st_ref = rest
        else:
            y_ref, st_ref = rest
        i = pl.program_id(0)
        x = x_ref[...]
        if pre_norm:
            st = st_in_ref[...]
            m = st[0:1] * (1.0 / _N)
            v = st[1:2] * (1.0 / _N) - m * m
            scale = gb_ref[0:1] * jax.lax.rsqrt(v + _EPS)
            shift = gb_ref[1:2] - m * scale
            e = _elu(x) * scale + shift
        else:
            e = x
        y = jnp.dot(e, w_ref[...], preferred_element_type=jnp.float32)
        y = y + b_ref[0:1, :]
        if add_res:
            y = y + res_ref[...]
        y_ref[...] = y
        ey = _elu(y)
        s = jnp.sum(ey, axis=0, keepdims=True)
        s2 = jnp.sum(ey * ey, axis=0, keepdims=True)
        st_new = jnp.concatenate(
            [s, s2, jnp.zeros((6, _C), jnp.float32)], axis=0)

        @pl.when(i == 0)
        def _init():
            st_ref[...] = jnp.zeros_like(st_ref)

        st_ref[...] += st_new

    return kfn


def _run_pass(x, w, b, st=None, gb=None, res=None):
    pre_norm = st is not None
    add_res = res is not None
    blk = pl.BlockSpec((_BLK, _C), lambda i: (i, 0))
    small = pl.BlockSpec((8, _C), lambda i: (0, 0))
    in_specs = [blk, pl.BlockSpec((_C, _C), lambda i: (0, 0)), small]
    ops = [x, w, jnp.broadcast_to(b[None, :], (8, _C))]
    if pre_norm:
        in_specs += [small, small]
        ops += [st, gb]
    if add_res:
        in_specs.append(blk)
        ops.append(res)
    y, st_out = pl.pallas_call(
        _make_pass(pre_norm, add_res),
        grid=(_N // _BLK,),
        in_specs=in_specs,
        out_specs=[blk, small],
        out_shape=[
            jax.ShapeDtypeStruct((_N, _C), jnp.float32),
            jax.ShapeDtypeStruct((8, _C), jnp.float32),
        ],
    )(*ops)
    return y, st_out


def _gb(gamma, beta):
    pad = jnp.zeros((6, _C), jnp.float32)
    return jnp.concatenate([gamma[None, :], beta[None, :], pad], axis=0)


def kernel(L, mask, inputs, W1, b1, rn, gamma2, beta2, W2, b2):
    del L, mask
    x0 = inputs.reshape(_N, _C)
    x, st = _run_pass(x0, W1, b1)
    res = x
    for i in range(4):
        for h in range(2):
            g = rn['gamma%d' % h][i]
            bt = rn['beta%d' % h][i]
            W = rn['W%d' % h][i]
            bb = rn['b%d' % h][i]
            b_eff = bt[_C:] @ W[_C:] + bb
            if h == 1:
                x, st = _run_pass(x, W[:_C], b_eff, st, _gb(g[:_C], bt[:_C]),
                                  res)
                res = x
            else:
                x, st = _run_pass(x, W[:_C], b_eff, st, _gb(g[:_C], bt[:_C]))
    y, _ = _run_pass(x, W2, b2, st, _gb(gamma2, beta2), x0)
    return y.reshape(1, _N, _C)


# speedup vs baseline: 2.3905x; 1.1715x over previous
import jax
import jax.numpy as jnp
from jax.experimental import pallas as pl

_N = 100000
_C = 128
_BLK = 10000
_EPS = 1e-5


def _elu(x):
    return jnp.where(x > 0, x, jnp.exp(jnp.minimum(x, 0.0)) - 1.0)


def _norm(h, st, gb):
    m = st[0:1] * (1.0 / _N)
    v = st[1:2] * (1.0 / _N) - m * m
    scale = gb[0:1] * jax.lax.rsqrt(v + _EPS)
    shift = gb[1:2] - m * scale
    return _elu(h) * scale + shift


def _make_pass(pre_norm_first, n_ops, res_mode, want_out, want_stats):
    def kfn(*refs):
        it = iter(refs)
        x_ref = next(it)
        w_ref = next(it)
        b_ref = next(it)
        st_ref = gb_ref = None
        if pre_norm_first or n_ops > 1:
            st_ref = next(it)
            gb_ref = next(it)
        res_ref = next(it) if res_mode == 'ext' else None
        y_ref = next(it) if want_out else None
        stout_ref = next(it) if want_stats else None
        i = pl.program_id(0)
        x = x_ref[...]
        h = x
        for j in range(n_ops):
            if j > 0 or pre_norm_first:
                h = _norm(h, st_ref[j], gb_ref[j])
            h = jnp.dot(h, w_ref[j], preferred_element_type=jnp.float32)
            h = h + b_ref[j, 0:1, :]
        if res_mode == 'self':
            h = h + x
        elif res_mode == 'ext':
            h = h + res_ref[...]
        if want_out:
            y_ref[...] = h
        if want_stats:
            ey = _elu(h)
            s = jnp.sum(ey, axis=0, keepdims=True)
            s2 = jnp.sum(ey * ey, axis=0, keepdims=True)
            st_new = jnp.concatenate(
                [s, s2, jnp.zeros((6, _C), jnp.float32)], axis=0)

            @pl.when(i == 0)
            def _init():
                stout_ref[...] = jnp.zeros_like(stout_ref)

            stout_ref[...] += st_new

    return kfn


def _run_pass(x, ws, bs, sts=None, gbs=None, res=None, *,
              pre_norm_first=True, want_out=True, want_stats=True):
    n_ops = len(ws)
    res_mode = None if res is None else ('self' if res is x else 'ext')
    blk = pl.BlockSpec((_BLK, _C), lambda i: (i, 0))
    small3 = pl.BlockSpec((n_ops, 8, _C), lambda i: (0, 0, 0))
    in_specs = [blk,
                pl.BlockSpec((n_ops, _C, _C), lambda i: (0, 0, 0)),
                small3]
    ops = [x, jnp.stack(ws),
           jnp.stack([jnp.broadcast_to(b[None, :], (8, _C)) for b in bs])]
    if sts is not None:
        in_specs += [small3, small3]
        ops += [jnp.stack(sts), jnp.stack(gbs)]
    if res_mode == 'ext':
        in_specs.append(blk)
        ops.append(res)
    out_specs, out_shape = [], []
    if want_out:
        out_specs.append(blk)
        out_shape.append(jax.ShapeDtypeStruct((_N, _C), jnp.float32))
    if want_stats:
        out_specs.append(pl.BlockSpec((8, _C), lambda i: (0, 0)))
        out_shape.append(jax.ShapeDtypeStruct((8, _C), jnp.float32))
    outs = pl.pallas_call(
        _make_pass(pre_norm_first, n_ops, res_mode, want_out, want_stats),
        grid=(_N // _BLK,),
        in_specs=in_specs,
        out_specs=out_specs,
        out_shape=out_shape,
    )(*ops)
    return outs


def _gb(gamma, beta):
    pad = jnp.zeros((6, _C), jnp.float32)
    return jnp.concatenate([gamma[None, :], beta[None, :], pad], axis=0)


def kernel(L, mask, inputs, W1, b1, rn, gamma2, beta2, W2, b2):
    del L, mask
    x0 = inputs.reshape(_N, _C)
    x, st = _run_pass(x0, [W1], [b1], pre_norm_first=False)
    for i in range(4):
        halves = []
        for h in range(2):
            g = rn['gamma%d' % h][i]
            bt = rn['beta%d' % h][i]
            W = rn['W%d' % h][i]
            bb = rn['b%d' % h][i]
            halves.append((W[:_C], bt[_C:] @ W[_C:] + bb,
                           _gb(g[:_C], bt[:_C])))
        (wa, ba, gba), (wb, bb2, gbb) = halves
        (st_a,) = _run_pass(x, [wa], [ba], [st], [gba],
                            want_out=False)
        x, st = _run_pass(x, [wa, wb], [ba, bb2], [st, st_a], [gba, gbb],
                          res=x)
    (y,) = _run_pass(x, [W2], [b2], [st], [_gb(gamma2, beta2)], res=x0,
                     want_stats=False)
    return y.reshape(1, _N, _C)


# speedup vs baseline: 2.4669x; 1.0319x over previous
import jax
import jax.numpy as jnp
from jax.experimental import pallas as pl

_N = 100000
_C = 128
_BLK = 10000
_EPS = 1e-5


def _elu(x):
    return jnp.where(x > 0, x, jnp.exp(x) - 1.0)


def _norm(h, st, gb):
    m = st[0:1] * (1.0 / _N)
    v = st[1:2] * (1.0 / _N) - m * m
    scale = gb[0:1] * jax.lax.rsqrt(v + _EPS)
    shift = gb[1:2] - m * scale
    return _elu(h) * scale + shift


def _make_pass(pre_norm_first, n_ops, res_mode, want_out, want_stats):
    def kfn(*refs):
        it = iter(refs)
        x_ref = next(it)
        w_ref = next(it)
        b_ref = next(it)
        st_ref = gb_ref = None
        if pre_norm_first or n_ops > 1:
            st_ref = next(it)
            gb_ref = next(it)
        res_ref = next(it) if res_mode == 'ext' else None
        y_ref = next(it) if want_out else None
        stout_ref = next(it) if want_stats else None
        i = pl.program_id(0)
        x = x_ref[...].astype(jnp.float32)
        h = x
        for j in range(n_ops):
            if j > 0 or pre_norm_first:
                h = _norm(h, st_ref[j], gb_ref[j])
            h = jnp.dot(h.astype(jnp.bfloat16), w_ref[j],
                        preferred_element_type=jnp.float32)
            h = h + b_ref[j, 0:1, :]
        if res_mode == 'self':
            h = h + x
        elif res_mode == 'ext':
            h = h + res_ref[...]
        if want_out:
            y_ref[...] = h.astype(y_ref.dtype)
        if want_stats:
            ey = _elu(h)
            s = jnp.sum(ey, axis=0, keepdims=True)
            s2 = jnp.sum(ey * ey, axis=0, keepdims=True)
            st_new = jnp.concatenate(
                [s, s2, jnp.zeros((6, _C), jnp.float32)], axis=0)

            @pl.when(i == 0)
            def _init():
                stout_ref[...] = jnp.zeros_like(stout_ref)

            stout_ref[...] += st_new

    return kfn


def _run_pass(x, ws, bs, sts=None, gbs=None, res=None, *,
              pre_norm_first=True, want_out=True, want_stats=True,
              out_dtype=jnp.bfloat16):
    n_ops = len(ws)
    res_mode = None if res is None else ('self' if res is x else 'ext')
    blk = pl.BlockSpec((_BLK, _C), lambda i: (i, 0))
    small3 = pl.BlockSpec((n_ops, 8, _C), lambda i: (0, 0, 0))
    in_specs = [blk,
                pl.BlockSpec((n_ops, _C, _C), lambda i: (0, 0, 0)),
                small3]
    ops = [x, jnp.stack(ws).astype(jnp.bfloat16),
           jnp.stack([jnp.broadcast_to(b[None, :], (8, _C)) for b in bs])]
    if sts is not None:
        in_specs += [small3, small3]
        ops += [jnp.stack(sts), jnp.stack(gbs)]
    if res_mode == 'ext':
        in_specs.append(blk)
        ops.append(res)
    out_specs, out_shape = [], []
    if want_out:
        out_specs.append(blk)
        out_shape.append(jax.ShapeDtypeStruct((_N, _C), out_dtype))
    if want_stats:
        out_specs.append(pl.BlockSpec((8, _C), lambda i: (0, 0)))
        out_shape.append(jax.ShapeDtypeStruct((8, _C), jnp.float32))
    outs = pl.pallas_call(
        _make_pass(pre_norm_first, n_ops, res_mode, want_out, want_stats),
        grid=(_N // _BLK,),
        in_specs=in_specs,
        out_specs=out_specs,
        out_shape=out_shape,
    )(*ops)
    return outs


def _gb(gamma, beta):
    pad = jnp.zeros((6, _C), jnp.float32)
    return jnp.concatenate([gamma[None, :], beta[None, :], pad], axis=0)


def kernel(L, mask, inputs, W1, b1, rn, gamma2, beta2, W2, b2):
    del L, mask
    x0 = inputs.reshape(_N, _C)
    x, st = _run_pass(x0, [W1], [b1], pre_norm_first=False)
    for i in range(4):
        halves = []
        for h in range(2):
            g = rn['gamma%d' % h][i]
            bt = rn['beta%d' % h][i]
            W = rn['W%d' % h][i]
            bb = rn['b%d' % h][i]
            halves.append((W[:_C], bt[_C:] @ W[_C:] + bb,
                           _gb(g[:_C], bt[:_C])))
        (wa, ba, gba), (wb, bb2, gbb) = halves
        (st_a,) = _run_pass(x, [wa], [ba], [st], [gba],
                            want_out=False)
        x, st = _run_pass(x, [wa, wb], [ba, bb2], [st, st_a], [gba, gbb],
                          res=x)
    (y,) = _run_pass(x, [W2], [b2], [st], [_gb(gamma2, beta2)], res=x0,
                     want_stats=False, out_dtype=jnp.float32)
    return y.reshape(1, _N, _C)


# speedup vs baseline: 2.7981x; 1.1343x over previous
import jax
import jax.numpy as jnp
from jax.experimental import pallas as pl

_N = 100000
_C = 128
_BLK = 10000
_EPS = 1e-5
_BF = jnp.bfloat16


def _elu(x):
    return jnp.where(x > 0, x, jnp.exp(x) - 1.0)


def _acc_stats(i, e32, st_ref):
    s = jnp.sum(e32, axis=0, keepdims=True)
    s2 = jnp.sum(e32 * e32, axis=0, keepdims=True)
    st = jnp.concatenate([s, s2, jnp.zeros((6, _C), jnp.float32)], axis=0)

    @pl.when(i == 0)
    def _init():
        st_ref[...] = jnp.zeros_like(st_ref)

    st_ref[...] += st


def _conv1_kernel(x_ref, w_ref, b_ref, xo_ref, eo_ref, st_ref):
    h = jnp.dot(x_ref[...].astype(_BF), w_ref[...],
                preferred_element_type=jnp.float32)
    h = h + b_ref[0:1, :]
    xo_ref[...] = h.astype(_BF)
    e = _elu(h)
    eo_ref[...] = e.astype(_BF)
    _acc_stats(pl.program_id(0), e, st_ref)


def _stats_kernel(e_ref, w_ref, b_ref, st_ref):
    h = jnp.dot(e_ref[...], w_ref[...], preferred_element_type=jnp.float32)
    h = h + b_ref[0:1, :]
    _acc_stats(pl.program_id(0), _elu(h), st_ref)


def _fused_kernel(e_ref, x_ref, wa_ref, ba_ref, wb_ref, bb_ref,
                  xo_ref, eo_ref, st_ref):
    ha = jnp.dot(e_ref[...], wa_ref[...], preferred_element_type=jnp.float32)
    ea = _elu(ha + ba_ref[0:1, :]).astype(_BF)
    hb = jnp.dot(ea, wb_ref[...], preferred_element_type=jnp.float32)
    hb = hb + bb_ref[0:1, :] + x_ref[...]
    xo_ref[...] = hb.astype(_BF)
    e = _elu(hb)
    eo_ref[...] = e.astype(_BF)
    _acc_stats(pl.program_id(0), e, st_ref)


def _final_kernel(e_ref, r_ref, w_ref, b_ref, y_ref):
    h = jnp.dot(e_ref[...], w_ref[...], preferred_element_type=jnp.float32)
    y_ref[...] = h + b_ref[0:1, :] + r_ref[...]


_BIG = pl.BlockSpec((_BLK, _C), lambda i: (i, 0))
_WSP = pl.BlockSpec((_C, _C), lambda i: (0, 0))
_SML = pl.BlockSpec((8, _C), lambda i: (0, 0))


def _call(kfn, ops, in_specs, outs):
    out_specs = [s for s, _ in outs]
    out_shape = [jax.ShapeDtypeStruct(sh, dt) for _, (sh, dt) in outs]
    return pl.pallas_call(kfn, grid=(_N // _BLK,), in_specs=in_specs,
                          out_specs=out_specs, out_shape=out_shape)(*ops)


def _b8(b):
    return jnp.broadcast_to(b[None, :], (8, _C))


def _fold(st, gamma_lo, beta_lo, w_lo, extra_b):
    m = st[0] * (1.0 / _N)
    v = st[1] * (1.0 / _N) - m * m
    scale = gamma_lo * jax.lax.rsqrt(v + _EPS)
    shift = beta_lo - m * scale
    return (scale[:, None] * w_lo).astype(_BF), shift @ w_lo + extra_b


_XE_ST = [(_BIG, ((_N, _C), _BF)), (_BIG, ((_N, _C), _BF)),
          (_SML, ((8, _C), jnp.float32))]


def kernel(L, mask, inputs, W1, b1, rn, gamma2, beta2, W2, b2):
    del L, mask
    x0f = inputs.reshape(_N, _C)
    x, e, st = _call(_conv1_kernel, [x0f, W1.astype(_BF), _b8(b1)],
                     [_BIG, _WSP, _SML], _XE_ST)
    for i in range(4):
        ws = []
        for h in range(2):
            g = rn['gamma%d' % h][i]
            bt = rn['beta%d' % h][i]
            W = rn['W%d' % h][i]
            bb = rn['b%d' % h][i]
            ws.append((g[:_C], bt[:_C], W[:_C], bt[_C:] @ W[_C:] + bb))
        wa, ba = _fold(st, *ws[0])
        (st_a,) = _call(_stats_kernel, [e, wa, _b8(ba)],
                        [_BIG, _WSP, _SML], [(_SML, ((8, _C), jnp.float32))])
        wb, bb_ = _fold(st_a, *ws[1])
        x, e, st = _call(_fused_kernel,
                         [e, x, wa, _b8(ba), wb, _b8(bb_)],
                         [_BIG, _BIG, _WSP, _SML, _WSP, _SML], _XE_ST)
    w2e, b2e = _fold(st, gamma2, beta2, W2, b2)
    (y,) = _call(_final_kernel, [e, x0f, w2e, _b8(b2e)],
                 [_BIG, _BIG, _WSP, _SML],
                 [(_BIG, ((_N, _C), jnp.float32))])
    return y.reshape(1, _N, _C)
